# baseline (speedup 1.0000x reference)
import jax
import jax.numpy as jnp
from jax import lax
from jax.experimental import pallas as pl
from jax.experimental.pallas import tpu as pltpu
from jax.experimental.pallas import tpu_sc as plsc

_N = 10000
_E = 320000
_NTILES = 16
_NCORES = 2
_CH = 128
_NCH = 160
_EPAD = _NTILES * _NCH * _CH
_NCH0 = 80
_IDXBLK = 40
_ZROWS = 40
_NPAD = 10240
_ZSTRIPE = _NPAD // _NTILES
_STRIPE = 1000
_CTILES = _N // _STRIPE
_BLK = 2000
_PH0 = 1


def _edge_range(c, s):
    wid = c * _NTILES + s
    per_row = _NCH // _NCH0
    return wid // per_row, (wid % per_row) * _NCH0


def _edge_loop_db(gc_hbm, src_v, dst_v, rows0, rows1, s_sh, sem0, sem1):
    def gather(j, buf, sem):
        return pltpu.make_async_copy(gc_hbm.at[src_v.at[j]], buf, sem)

    gather(0, rows0, sem0).start()

    def body2(j2, _):
        b = j2 * 2
        gather(b + 1, rows1, sem1).start()
        gather(b, rows0, sem0).wait()
        pltpu.sync_copy(rows0, s_sh.at[dst_v.at[b]], add=True)

        @pl.when(b + 2 < _IDXBLK)
        def _():
            gather(b + 2, rows0, sem0).start()
        gather(b + 1, rows1, sem1).wait()
        pltpu.sync_copy(rows1, s_sh.at[dst_v.at[b + 1]], add=True)
        return 0
    lax.fori_loop(0, _IDXBLK // 2, body2, 0)


def _deg_body(dstx_hbm, out_hbm, dst_v, zbuf_v, ones_v, s_sh, sem):
    c = lax.axis_index("c")
    s = lax.axis_index("s")

    def zero_row(t, _):
        zbuf_v[t // 8, pl.ds((t % 8) * 16, 16)] = (
            jnp.zeros((16,), jnp.float32))
        return 0
    lax.fori_loop(0, _ZROWS * 8, zero_row, 0)

    @pl.when(s < _CTILES)
    def _():
        ibase = s * _STRIPE
        for k in range(_STRIPE // _ZROWS):
            pltpu.sync_copy(
                zbuf_v, s_sh.at[pl.ds(ibase + k * _ZROWS, _ZROWS)])

    @pl.when(s == _CTILES)
    def _():
        for k in range((_NPAD - _N) // _ZROWS):
            pltpu.sync_copy(
                zbuf_v, s_sh.at[pl.ds(_N + k * _ZROWS, _ZROWS)])

    def one_row(t, _):
        ones_v[t // 8, pl.ds((t % 8) * 16, 16)] = (
            jnp.ones((16,), jnp.float32))
        return 0
    lax.fori_loop(0, _CH * 8, one_row, 0)
    plsc.subcore_barrier()

    row, off = _edge_range(c, s)
    for p in range(_NCH0 // _IDXBLK):
        pltpu.sync_copy(dstx_hbm.at[row, pl.ds(off + p * _IDXBLK, _IDXBLK)],
                        dst_v)

        def body(j, _):
            pltpu.sync_copy(ones_v, s_sh.at[dst_v.at[j]], add=True)
            return 0
        lax.fori_loop(0, _IDXBLK, body, 0)

    plsc.subcore_barrier()

    @pl.when(s < _CTILES)
    def _():
        obase = s * _STRIPE
        pltpu.sync_copy(s_sh.at[pl.ds(obase, _STRIPE)],
                        out_hbm.at[c, pl.ds(obase, _STRIPE)])


def _make_deg_kernel():
    mesh = plsc.VectorSubcoreMesh(core_axis_name="c", subcore_axis_name="s",
                                  num_cores=_NCORES)
    return pl.kernel(
        _deg_body,
        out_type=jax.ShapeDtypeStruct((_NCORES, _N, 128), jnp.float32),
        mesh=mesh,
        scratch_types=[
            pltpu.VMEM((_IDXBLK, _CH), jnp.int32),
            pltpu.VMEM((_ZROWS, 128), jnp.float32),
            pltpu.VMEM((_CH, 128), jnp.float32),
            pltpu.VMEM_SHARED((_NPAD, 128), jnp.float32),
            pltpu.SemaphoreType.DMA,
        ],
    )


def _scatter0_body(g_hbm, srcx_hbm, dstx_hbm, out_hbm,
                   src_v, dst_v, rows0_v, rows1_v, zbuf_v, s_sh,
                   sem0, sem1):
    c = lax.axis_index("c")
    s = lax.axis_index("s")

    @pl.when((s < _CTILES) & (c == 0))
    def _():
        ibase = s * _STRIPE
        pltpu.sync_copy(g_hbm.at[pl.ds(ibase, _STRIPE)],
                        s_sh.at[pl.ds(ibase, _STRIPE)])

    @pl.when((s < _CTILES) & (c == 1))
    def _():
        def zero_row(t, _):
            zbuf_v[t // 8, pl.ds((t % 8) * 16, 16)] = (
                jnp.zeros((16,), jnp.float32))
            return 0
        lax.fori_loop(0, _ZROWS * 8, zero_row, 0)
        ibase = s * _STRIPE
        for k in range(_STRIPE // _ZROWS):
            pltpu.sync_copy(
                zbuf_v, s_sh.at[pl.ds(ibase + k * _ZROWS, _ZROWS)])

    plsc.subcore_barrier()

    for p in range(_NCH // _IDXBLK):
        cond = (c == 0) if p < _PH0 else (c == 1)

        @pl.when(cond)
        def _():
            pltpu.sync_copy(srcx_hbm.at[s, pl.ds(p * _IDXBLK, _IDXBLK)],
                            src_v)
            pltpu.sync_copy(dstx_hbm.at[s, pl.ds(p * _IDXBLK, _IDXBLK)],
                            dst_v)
            _edge_loop_db(g_hbm, src_v, dst_v, rows0_v, rows1_v, s_sh,
                          sem0, sem1)

    plsc.subcore_barrier()

    @pl.when(s < _CTILES)
    def _():
        obase = s * _STRIPE
        pltpu.sync_copy(s_sh.at[pl.ds(obase, _STRIPE)],
                        out_hbm.at[c, pl.ds(obase, _STRIPE)])


def _make_scatter0_kernel():
    mesh = plsc.VectorSubcoreMesh(core_axis_name="c", subcore_axis_name="s",
                                  num_cores=_NCORES)
    return pl.kernel(
        _scatter0_body,
        out_type=jax.ShapeDtypeStruct((_NCORES, _N, 128), jnp.float32),
        mesh=mesh,
        scratch_types=[
            pltpu.VMEM((_IDXBLK, _CH), jnp.int32),
            pltpu.VMEM((_IDXBLK, _CH), jnp.int32),
            pltpu.VMEM((_CH, 128), jnp.float32),
            pltpu.VMEM((_CH, 128), jnp.float32),
            pltpu.VMEM((_ZROWS, 128), jnp.float32),
            pltpu.VMEM_SHARED((_NPAD, 128), jnp.float32),
            pltpu.SemaphoreType.DMA,
            pltpu.SemaphoreType.DMA,
        ],
    )


def _scatter1_body(g3_hbm, srcx_hbm, dstx_hbm, out_hbm,
                   src_v, dst_v, rows0_v, rows1_v, s_sh, sem0, sem1):
    c = lax.axis_index("c")
    s = lax.axis_index("s")
    gc_hbm = g3_hbm.at[c]

    @pl.when(s < _CTILES)
    def _():
        ibase = s * _STRIPE
        pltpu.sync_copy(gc_hbm.at[pl.ds(ibase, _STRIPE)],
                        s_sh.at[pl.ds(ibase, _STRIPE)])

    plsc.subcore_barrier()

    for p in range(_NCH // _IDXBLK):
        pltpu.sync_copy(srcx_hbm.at[s, pl.ds(p * _IDXBLK, _IDXBLK)], src_v)
        pltpu.sync_copy(dstx_hbm.at[s, pl.ds(p * _IDXBLK, _IDXBLK)], dst_v)

        _edge_loop_db(gc_hbm, src_v, dst_v, rows0_v, rows1_v, s_sh,
                      sem0, sem1)

    plsc.subcore_barrier()

    @pl.when(s < _CTILES)
    def _():
        obase = s * _STRIPE
        pltpu.sync_copy(s_sh.at[pl.ds(obase, _STRIPE)],
                        out_hbm.at[c, pl.ds(obase, _STRIPE)])


def _make_scatter1_kernel():
    mesh = plsc.VectorSubcoreMesh(core_axis_name="c", subcore_axis_name="s",
                                  num_cores=_NCORES)
    return pl.kernel(
        _scatter1_body,
        out_type=jax.ShapeDtypeStruct((_NCORES, _N, 128), jnp.float32),
        mesh=mesh,
        scratch_types=[
            pltpu.VMEM((_IDXBLK, _CH), jnp.int32),
            pltpu.VMEM((_IDXBLK, _CH), jnp.int32),
            pltpu.VMEM((_CH, 128), jnp.float32),
            pltpu.VMEM((_CH, 128), jnp.float32),
            pltpu.VMEM_SHARED((_NPAD, 128), jnp.float32),
            pltpu.SemaphoreType.DMA,
            pltpu.SemaphoreType.DMA,
        ],
    )


def _dinv_of(d0, d1):
    return lax.rsqrt(d0[:, 0:1] + d1[:, 0:1] + 1.0)


def _mm0_body(x_ref, w_ref, d0_ref, d1_ref, o_ref):
    dinv = _dinv_of(d0_ref[0], d1_ref[0])
    o_ref[...] = jnp.dot(x_ref[...], w_ref[...],
                         preferred_element_type=jnp.float32) * dinv


def _mm1_body(p0_ref, p1_ref, d0_ref, d1_ref, w_ref, b_ref, o_ref):
    dinv = _dinv_of(d0_ref[0], d1_ref[0])
    h0 = jnp.maximum((p0_ref[0] + p1_ref[0]) * dinv + b_ref[...], 0.0)
    g = jnp.dot(h0, w_ref[...], preferred_element_type=jnp.float32) * dinv
    o_ref[0] = g[:, :128]
    o_ref[1] = g[:, 128:]


def _head_body(sl_ref, sr_ref, d0_ref, d1_ref, b1_ref, wl1_ref, bl1_ref,
               wl2_ref, wsex_ref, wcag_ref, bl2_ref, sex_ref, cag_ref,
               pred_ref, xlin_ref):
    dinv = _dinv_of(d0_ref[0], d1_ref[0])
    ssum = jnp.concatenate([sl_ref[0], sr_ref[0]], axis=1)
    h1 = jnp.maximum(ssum * dinv + b1_ref[...], 0.0)
    xl = jnp.maximum(
        jnp.dot(h1, wl1_ref[...], preferred_element_type=jnp.float32)
        + bl1_ref[...], 0.0)
    xlin_ref[...] = xl
    logits = (jnp.dot(xl, wl2_ref[...], preferred_element_type=jnp.float32)
              + sex_ref[...] * wsex_ref[...]
              + cag_ref[...] * wcag_ref[...] + bl2_ref[...])
    m = jnp.max(logits, axis=1, keepdims=True)
    e = jnp.exp(logits - m)
    pred_ref[...] = e / jnp.sum(e, axis=1, keepdims=True)


def _full(shape):
    return pl.BlockSpec(shape, lambda i: tuple(0 for _ in shape))


def _dspec():
    return [pl.BlockSpec((1, _BLK, 128), lambda i: (0, i, 0)),
            pl.BlockSpec((1, _BLK, 128), lambda i: (1, i, 0))]


def kernel(x, edge_index, batch, sex, cag, W0, b0, Wg0, bg0, W1, b1, Wg1,
           bg1, Wl1, bl1, Wl2, bl2):
    f32 = jnp.float32
    nblk = _N // _BLK

    src = edge_index[0]
    dst = edge_index[1]
    pad = _EPAD - _E
    srcx = jnp.concatenate(
        [src, jnp.zeros((pad,), jnp.int32)]).reshape(_NTILES, _NCH, _CH)
    dstx = jnp.concatenate(
        [dst, jnp.full((pad,), _N, jnp.int32)]).reshape(_NTILES, _NCH, _CH)

    degp = _make_deg_kernel()(dstx)

    g0 = pl.pallas_call(
        _mm0_body,
        grid=(nblk,),
        in_specs=[
            pl.BlockSpec((_BLK, 128), lambda i: (i, 0)),
            _full((128, 128)),
            *_dspec(),
        ],
        out_specs=pl.BlockSpec((_BLK, 128), lambda i: (i, 0)),
        out_shape=jax.ShapeDtypeStruct((_N, 128), f32),
    )(x, W0, degp, degp)

    s0 = _make_scatter0_kernel()(g0, srcx, dstx)

    g1 = pl.pallas_call(
        _mm1_body,
        grid=(nblk,),
        in_specs=[
            pl.BlockSpec((1, _BLK, 128), lambda i: (0, i, 0)),
            pl.BlockSpec((1, _BLK, 128), lambda i: (1, i, 0)),
            *_dspec(),
            _full((128, 256)),
            _full((1, 128)),
        ],
        out_specs=pl.BlockSpec((_NCORES, _BLK, 128), lambda i: (0, i, 0)),
        out_shape=jax.ShapeDtypeStruct((_NCORES, _N, 128), f32),
    )(s0, s0, degp, degp, W1, b0.reshape(1, 128))

    s1 = _make_scatter1_kernel()(g1, srcx, dstx)

    pred, xlin = pl.pallas_call(
        _head_body,
        grid=(nblk,),
        in_specs=[
            pl.BlockSpec((1, _BLK, 128), lambda i: (0, i, 0)),
            pl.BlockSpec((1, _BLK, 128), lambda i: (1, i, 0)),
            *_dspec(),
            _full((1, 256)),
            _full((256, 256)),
            _full((1, 256)),
            _full((256, 5)),
            _full((1, 5)),
            _full((1, 5)),
            _full((1, 5)),
            pl.BlockSpec((_BLK, 1), lambda i: (i, 0)),
            pl.BlockSpec((_BLK, 1), lambda i: (i, 0)),
        ],
        out_specs=[
            pl.BlockSpec((_BLK, 5), lambda i: (i, 0)),
            pl.BlockSpec((_BLK, 256), lambda i: (i, 0)),
        ],
        out_shape=[
            jax.ShapeDtypeStruct((_N, 5), f32),
            jax.ShapeDtypeStruct((_N, 256), f32),
        ],
    )(s1, s1, degp, degp, b1.reshape(1, 256), Wl1, bl1.reshape(1, 256),
      Wl2[:256], Wl2[256:257], Wl2[257:258], bl2.reshape(1, 5),
      sex.reshape(_N, 1), cag.reshape(_N, 1))

    return (pred, xlin)


# speedup vs baseline: 1.0809x; 1.0809x over previous
import jax
import jax.numpy as jnp
from jax import lax
from jax.experimental import pallas as pl
from jax.experimental.pallas import tpu as pltpu
from jax.experimental.pallas import tpu_sc as plsc

_N = 10000
_E = 320000
_NTILES = 16
_NCORES = 2
_CH = 128
_NCH = 160
_EPAD = _NTILES * _NCH * _CH
_NCH0 = 80
_IDXBLK = 40
_ZROWS = 40
_NPAD = 10240
_ZSTRIPE = _NPAD // _NTILES
_STRIPE = 1000
_CTILES = _N // _STRIPE
_BLK = 2000
_PH0 = 3


def _edge_range(c, s):
    wid = c * _NTILES + s
    per_row = _NCH // _NCH0
    return wid // per_row, (wid % per_row) * _NCH0


def _edge_loop_db(gc_hbm, src_v, dst_v, rows0, rows1, s_sh, sem0, sem1):
    def gather(j, buf, sem):
        return pltpu.make_async_copy(gc_hbm.at[src_v.at[j]], buf, sem)

    gather(0, rows0, sem0).start()

    def body2(j2, _):
        b = j2 * 2
        gather(b + 1, rows1, sem1).start()
        gather(b, rows0, sem0).wait()
        pltpu.sync_copy(rows0, s_sh.at[dst_v.at[b]], add=True)

        @pl.when(b + 2 < _IDXBLK)
        def _():
            gather(b + 2, rows0, sem0).start()
        gather(b + 1, rows1, sem1).wait()
        pltpu.sync_copy(rows1, s_sh.at[dst_v.at[b + 1]], add=True)
        return 0
    lax.fori_loop(0, _IDXBLK // 2, body2, 0)


def _deg_body(dstx_hbm, out_hbm, dst_v, zbuf_v, ones_v, s_sh, sem):
    c = lax.axis_index("c")
    s = lax.axis_index("s")

    def zero_row(t, _):
        zbuf_v[t // 8, pl.ds((t % 8) * 16, 16)] = (
            jnp.zeros((16,), jnp.float32))
        return 0
    lax.fori_loop(0, _ZROWS * 8, zero_row, 0)

    @pl.when(s < _CTILES)
    def _():
        ibase = s * _STRIPE
        for k in range(_STRIPE // _ZROWS):
            pltpu.sync_copy(
                zbuf_v, s_sh.at[pl.ds(ibase + k * _ZROWS, _ZROWS)])

    @pl.when(s == _CTILES)
    def _():
        for k in range((_NPAD - _N) // _ZROWS):
            pltpu.sync_copy(
                zbuf_v, s_sh.at[pl.ds(_N + k * _ZROWS, _ZROWS)])

    def one_row(t, _):
        ones_v[t // 8, pl.ds((t % 8) * 16, 16)] = (
            jnp.ones((16,), jnp.float32))
        return 0
    lax.fori_loop(0, _CH * 8, one_row, 0)
    plsc.subcore_barrier()

    row, off = _edge_range(c, s)
    for p in range(_NCH0 // _IDXBLK):
        pltpu.sync_copy(dstx_hbm.at[row, pl.ds(off + p * _IDXBLK, _IDXBLK)],
                        dst_v)

        def body(j, _):
            pltpu.sync_copy(ones_v, s_sh.at[dst_v.at[j]], add=True)
            return 0
        lax.fori_loop(0, _IDXBLK, body, 0)

    plsc.subcore_barrier()

    @pl.when(s < _CTILES)
    def _():
        obase = s * _STRIPE
        pltpu.sync_copy(s_sh.at[pl.ds(obase, _STRIPE)],
                        out_hbm.at[c, pl.ds(obase, _STRIPE)])


def _make_deg_kernel():
    mesh = plsc.VectorSubcoreMesh(core_axis_name="c", subcore_axis_name="s",
                                  num_cores=_NCORES)
    return pl.kernel(
        _deg_body,
        out_type=jax.ShapeDtypeStruct((_NCORES, _N, 128), jnp.float32),
        mesh=mesh,
        scratch_types=[
            pltpu.VMEM((_IDXBLK, _CH), jnp.int32),
            pltpu.VMEM((_ZROWS, 128), jnp.float32),
            pltpu.VMEM((_CH, 128), jnp.float32),
            pltpu.VMEM_SHARED((_NPAD, 128), jnp.float32),
            pltpu.SemaphoreType.DMA,
        ],
    )


def _scatter0_body(g_hbm, srcx_hbm, dstx_hbm, out_hbm,
                   src_v, dst_v, rows0_v, rows1_v, zbuf_v, s_sh,
                   sem0, sem1):
    c = lax.axis_index("c")
    s = lax.axis_index("s")

    @pl.when((s < _CTILES) & (c == 0))
    def _():
        ibase = s * _STRIPE
        pltpu.sync_copy(g_hbm.at[pl.ds(ibase, _STRIPE)],
                        s_sh.at[pl.ds(ibase, _STRIPE)])

    @pl.when((s < _CTILES) & (c == 1))
    def _():
        def zero_row(t, _):
            zbuf_v[t // 8, pl.ds((t % 8) * 16, 16)] = (
                jnp.zeros((16,), jnp.float32))
            return 0
        lax.fori_loop(0, _ZROWS * 8, zero_row, 0)
        ibase = s * _STRIPE
        for k in range(_STRIPE // _ZROWS):
            pltpu.sync_copy(
                zbuf_v, s_sh.at[pl.ds(ibase + k * _ZROWS, _ZROWS)])

    plsc.subcore_barrier()

    for p in range(_NCH // _IDXBLK):
        cond = (c == 0) if p < _PH0 else (c == 1)

        @pl.when(cond)
        def _():
            pltpu.sync_copy(srcx_hbm.at[s, pl.ds(p * _IDXBLK, _IDXBLK)],
                            src_v)
            pltpu.sync_copy(dstx_hbm.at[s, pl.ds(p * _IDXBLK, _IDXBLK)],
                            dst_v)
            _edge_loop_db(g_hbm, src_v, dst_v, rows0_v, rows1_v, s_sh,
                          sem0, sem1)

    plsc.subcore_barrier()

    @pl.when(s < _CTILES)
    def _():
        obase = s * _STRIPE
        pltpu.sync_copy(s_sh.at[pl.ds(obase, _STRIPE)],
                        out_hbm.at[c, pl.ds(obase, _STRIPE)])


def _make_scatter0_kernel():
    mesh = plsc.VectorSubcoreMesh(core_axis_name="c", subcore_axis_name="s",
                                  num_cores=_NCORES)
    return pl.kernel(
        _scatter0_body,
        out_type=jax.ShapeDtypeStruct((_NCORES, _N, 128), jnp.float32),
        mesh=mesh,
        scratch_types=[
            pltpu.VMEM((_IDXBLK, _CH), jnp.int32),
            pltpu.VMEM((_IDXBLK, _CH), jnp.int32),
            pltpu.VMEM((_CH, 128), jnp.float32),
            pltpu.VMEM((_CH, 128), jnp.float32),
            pltpu.VMEM((_ZROWS, 128), jnp.float32),
            pltpu.VMEM_SHARED((_NPAD, 128), jnp.float32),
            pltpu.SemaphoreType.DMA,
            pltpu.SemaphoreType.DMA,
        ],
    )


def _scatter1_body(g3_hbm, srcx_hbm, dstx_hbm, out_hbm,
                   src_v, dst_v, rows0_v, rows1_v, s_sh, sem0, sem1):
    c = lax.axis_index("c")
    s = lax.axis_index("s")
    gc_hbm = g3_hbm.at[c]

    @pl.when(s < _CTILES)
    def _():
        ibase = s * _STRIPE
        pltpu.sync_copy(gc_hbm.at[pl.ds(ibase, _STRIPE)],
                        s_sh.at[pl.ds(ibase, _STRIPE)])

    plsc.subcore_barrier()

    for p in range(_NCH // _IDXBLK):
        pltpu.sync_copy(srcx_hbm.at[s, pl.ds(p * _IDXBLK, _IDXBLK)], src_v)
        pltpu.sync_copy(dstx_hbm.at[s, pl.ds(p * _IDXBLK, _IDXBLK)], dst_v)

        _edge_loop_db(gc_hbm, src_v, dst_v, rows0_v, rows1_v, s_sh,
                      sem0, sem1)

    plsc.subcore_barrier()

    @pl.when(s < _CTILES)
    def _():
        obase = s * _STRIPE
        pltpu.sync_copy(s_sh.at[pl.ds(obase, _STRIPE)],
                        out_hbm.at[c, pl.ds(obase, _STRIPE)])


def _make_scatter1_kernel():
    mesh = plsc.VectorSubcoreMesh(core_axis_name="c", subcore_axis_name="s",
                                  num_cores=_NCORES)
    return pl.kernel(
        _scatter1_body,
        out_type=jax.ShapeDtypeStruct((_NCORES, _N, 128), jnp.float32),
        mesh=mesh,
        scratch_types=[
            pltpu.VMEM((_IDXBLK, _CH), jnp.int32),
            pltpu.VMEM((_IDXBLK, _CH), jnp.int32),
            pltpu.VMEM((_CH, 128), jnp.float32),
            pltpu.VMEM((_CH, 128), jnp.float32),
            pltpu.VMEM_SHARED((_NPAD, 128), jnp.float32),
            pltpu.SemaphoreType.DMA,
            pltpu.SemaphoreType.DMA,
        ],
    )


def _dinv_of(d0, d1):
    return lax.rsqrt(d0[:, 0:1] + d1[:, 0:1] + 1.0)


def _mm0_body(x_ref, w_ref, d0_ref, d1_ref, o_ref):
    dinv = _dinv_of(d0_ref[0], d1_ref[0])
    o_ref[...] = jnp.dot(x_ref[...], w_ref[...],
                         preferred_element_type=jnp.float32) * dinv


def _mm1_body(p0_ref, p1_ref, d0_ref, d1_ref, w_ref, b_ref, o_ref):
    dinv = _dinv_of(d0_ref[0], d1_ref[0])
    h0 = jnp.maximum((p0_ref[0] + p1_ref[0]) * dinv + b_ref[...], 0.0)
    g = jnp.dot(h0, w_ref[...], preferred_element_type=jnp.float32) * dinv
    o_ref[0] = g[:, :128]
    o_ref[1] = g[:, 128:]


def _head_body(sl_ref, sr_ref, d0_ref, d1_ref, b1_ref, wl1_ref, bl1_ref,
               wl2_ref, wsex_ref, wcag_ref, bl2_ref, sex_ref, cag_ref,
               pred_ref, xlin_ref):
    dinv = _dinv_of(d0_ref[0], d1_ref[0])
    ssum = jnp.concatenate([sl_ref[0], sr_ref[0]], axis=1)
    h1 = jnp.maximum(ssum * dinv + b1_ref[...], 0.0)
    xl = jnp.maximum(
        jnp.dot(h1, wl1_ref[...], preferred_element_type=jnp.float32)
        + bl1_ref[...], 0.0)
    xlin_ref[...] = xl
    logits = (jnp.dot(xl, wl2_ref[...], preferred_element_type=jnp.float32)
              + sex_ref[...] * wsex_ref[...]
              + cag_ref[...] * wcag_ref[...] + bl2_ref[...])
    m = jnp.max(logits, axis=1, keepdims=True)
    e = jnp.exp(logits - m)
    pred_ref[...] = e / jnp.sum(e, axis=1, keepdims=True)


def _full(shape):
    return pl.BlockSpec(shape, lambda i: tuple(0 for _ in shape))


def _dspec():
    return [pl.BlockSpec((1, _BLK, 128), lambda i: (0, i, 0)),
            pl.BlockSpec((1, _BLK, 128), lambda i: (1, i, 0))]


def kernel(x, edge_index, batch, sex, cag, W0, b0, Wg0, bg0, W1, b1, Wg1,
           bg1, Wl1, bl1, Wl2, bl2):
    f32 = jnp.float32
    nblk = _N // _BLK

    src = edge_index[0]
    dst = edge_index[1]
    pad = _EPAD - _E
    srcx = jnp.concatenate(
        [src, jnp.zeros((pad,), jnp.int32)]).reshape(_NTILES, _NCH, _CH)
    dstx = jnp.concatenate(
        [dst, jnp.full((pad,), _N, jnp.int32)]).reshape(_NTILES, _NCH, _CH)

    degp = _make_deg_kernel()(dstx)

    g0 = pl.pallas_call(
        _mm0_body,
        grid=(nblk,),
        in_specs=[
            pl.BlockSpec((_BLK, 128), lambda i: (i, 0)),
            _full((128, 128)),
            *_dspec(),
        ],
        out_specs=pl.BlockSpec((_BLK, 128), lambda i: (i, 0)),
        out_shape=jax.ShapeDtypeStruct((_N, 128), f32),
    )(x, W0, degp, degp)

    s0 = _make_scatter0_kernel()(g0, srcx, dstx)

    g1 = pl.pallas_call(
        _mm1_body,
        grid=(nblk,),
        in_specs=[
            pl.BlockSpec((1, _BLK, 128), lambda i: (0, i, 0)),
            pl.BlockSpec((1, _BLK, 128), lambda i: (1, i, 0)),
            *_dspec(),
            _full((128, 256)),
            _full((1, 128)),
        ],
        out_specs=pl.BlockSpec((_NCORES, _BLK, 128), lambda i: (0, i, 0)),
        out_shape=jax.ShapeDtypeStruct((_NCORES, _N, 128), f32),
    )(s0, s0, degp, degp, W1, b0.reshape(1, 128))

    s1 = _make_scatter1_kernel()(g1, srcx, dstx)

    pred, xlin = pl.pallas_call(
        _head_body,
        grid=(nblk,),
        in_specs=[
            pl.BlockSpec((1, _BLK, 128), lambda i: (0, i, 0)),
            pl.BlockSpec((1, _BLK, 128), lambda i: (1, i, 0)),
            *_dspec(),
            _full((1, 256)),
            _full((256, 256)),
            _full((1, 256)),
            _full((256, 5)),
            _full((1, 5)),
            _full((1, 5)),
            _full((1, 5)),
            pl.BlockSpec((_BLK, 1), lambda i: (i, 0)),
            pl.BlockSpec((_BLK, 1), lambda i: (i, 0)),
        ],
        out_specs=[
            pl.BlockSpec((_BLK, 5), lambda i: (i, 0)),
            pl.BlockSpec((_BLK, 256), lambda i: (i, 0)),
        ],
        out_shape=[
            jax.ShapeDtypeStruct((_N, 5), f32),
            jax.ShapeDtypeStruct((_N, 256), f32),
        ],
    )(s1, s1, degp, degp, b1.reshape(1, 256), Wl1, bl1.reshape(1, 256),
      Wl2[:256], Wl2[256:257], Wl2[257:258], bl2.reshape(1, 5),
      sex.reshape(_N, 1), cag.reshape(_N, 1))

    return (pred, xlin)
